# baseline (speedup 1.0000x reference)
import functools

import jax
import jax.numpy as jnp
from jax import lax
from jax.experimental import pallas as pl
from jax.experimental.pallas import tpu as pltpu
from jax.experimental.pallas import tpu_sc as plsc

_SL, _B, _V, _D = 200, 4096, 1000000, 64
_NW = 32
_N = _SL * _B
_PER_W = _N // _NW
_C = 128
_NCH = _PER_W // _C
_L = 16

_mesh = plsc.VectorSubcoreMesh(core_axis_name="c", subcore_axis_name="s")


@functools.partial(
    pl.kernel,
    mesh=_mesh,
    compiler_params=pltpu.CompilerParams(use_tc_tiling_on_sc=False),
    out_type=jax.ShapeDtypeStruct((_N, _D), jnp.float32),
    scratch_types=[
        pltpu.VMEM((_PER_W,), jnp.int32),
        pltpu.VMEM((_SL, _D), jnp.float32),
        pltpu.VMEM((_C, _D), jnp.float32),
        pltpu.SemaphoreType.DMA,
    ],
)
def _embed(x_hbm, tok_hbm, pos_hbm, out_hbm, idx_v, pos_v, rows_v, sem):
    wid = lax.axis_index("s") * 2 + lax.axis_index("c")
    base = wid * _PER_W
    pltpu.sync_copy(x_hbm.at[pl.ds(base, _PER_W)], idx_v)
    pltpu.sync_copy(pos_hbm, pos_v)

    def chunk_body(g, carry):
        start = base + g * _C
        sl = start // _B
        pltpu.async_copy(
            tok_hbm.at[idx_v.at[pl.ds(g * _C, _C)]], rows_v, sem
        ).wait()
        pvecs = [pos_v[sl, pl.ds(j * _L, _L)] for j in range(_D // _L)]

        def row_body(r, c2):
            for j in range(_D // _L):
                plsc.addupdate(rows_v.at[r, pl.ds(j * _L, _L)], pvecs[j])
            return c2

        lax.fori_loop(0, _C, row_body, 0)
        pltpu.sync_copy(rows_v, out_hbm.at[pl.ds(start, _C)])
        return carry

    lax.fori_loop(0, _NCH, chunk_body, 0)


def kernel(x, token_table, pos_table):
    out = _embed(x.reshape(_N), token_table, pos_table)
    return out.reshape(_SL, _B, _D)


# speedup vs baseline: 1.2037x; 1.2037x over previous
import functools

import jax
import jax.numpy as jnp
from jax import lax
from jax.experimental import pallas as pl
from jax.experimental.pallas import tpu as pltpu
from jax.experimental.pallas import tpu_sc as plsc

_SL, _B, _V, _D = 200, 4096, 1000000, 64
_NW = 32
_N = _SL * _B
_PER_W = _N // _NW
_C = 128
_NCH = _PER_W // _C
_L = 16
_NB = 4
_RU = 4

_mesh = plsc.VectorSubcoreMesh(core_axis_name="c", subcore_axis_name="s")


@functools.partial(
    pl.kernel,
    mesh=_mesh,
    compiler_params=pltpu.CompilerParams(use_tc_tiling_on_sc=False),
    out_type=jax.ShapeDtypeStruct((_N, _D), jnp.float32),
    scratch_types=[
        pltpu.VMEM((_PER_W,), jnp.int32),
        pltpu.VMEM((_SL, _D), jnp.float32),
        pltpu.VMEM((_NB, _C, _D), jnp.float32),
        pltpu.SemaphoreType.DMA((_NB,)),
        pltpu.SemaphoreType.DMA((_NB,)),
    ],
)
def _embed(x_hbm, tok_hbm, pos_hbm, out_hbm, idx_v, pos_v, bufs, gsem, osem):
    wid = lax.axis_index("s") * 2 + lax.axis_index("c")
    base = wid * _PER_W
    pltpu.sync_copy(x_hbm.at[pl.ds(base, _PER_W)], idx_v)
    pltpu.sync_copy(pos_hbm, pos_v)

    def gather(g, b):
        return pltpu.make_async_copy(
            tok_hbm.at[idx_v.at[pl.ds(g * _C, _C)]], bufs.at[b], gsem.at[b]
        )

    def store(g, b):
        return pltpu.make_async_copy(
            bufs.at[b], out_hbm.at[pl.ds(base + g * _C, _C)], osem.at[b]
        )

    for b in range(_NB):
        gather(b, b).start()

    def group_body(i, carry):
        g0 = i * _NB
        for b in range(_NB):
            g = g0 + b
            gather(g, b).wait()
            sl = (base + g * _C) // _B
            pvecs = [pos_v[sl, pl.ds(j * _L, _L)] for j in range(_D // _L)]

            def row_body(r, c2, b=b, pvecs=pvecs):
                r0 = r * _RU
                for u in range(_RU):
                    for j in range(_D // _L):
                        plsc.addupdate(
                            bufs.at[b, r0 + u, pl.ds(j * _L, _L)], pvecs[j]
                        )
                return c2

            lax.fori_loop(0, _C // _RU, row_body, 0, unroll=1)
            store(g, b).start()

            bp = (b - 1) % _NB
            gp = g - 1

            @pl.when((gp >= 0) & (gp + _NB < _NCH))
            def _():
                store(gp, bp).wait()
                gather(gp + _NB, bp).start()

        return carry

    lax.fori_loop(0, _NCH // _NB, group_body, 0)

    for b in range(_NB):
        store(_NCH - _NB + b, b).wait()


def kernel(x, token_table, pos_table):
    out = _embed(x.reshape(_N), token_table, pos_table)
    return out.reshape(_SL, _B, _D)
